# Initial kernel scaffold; baseline (speedup 1.0000x reference)
import jax
import jax.numpy as jnp
from jax.experimental import pallas as pl


def kernel(positions, cells, numbers, edge_indices, edge_offsets, batch, W_species, ln_gamma, ln_beta, Wc1, a_src1, a_dst1, Wc2, a_src2, a_dst2, Wn, Wo):
    raise NotImplementedError("write your pallas kernel here")


# speedup vs baseline: 16.8937x; 16.8937x over previous
import functools

import jax
import jax.numpy as jnp
from jax import lax
from jax.experimental import pallas as pl
from jax.experimental.pallas import tpu as pltpu
from jax.experimental.pallas import tpu_sc as plsc

NC = 2
NS = 16
CHUNK = 128

N = 50000
E = 800000
B = 500
C = 4
R = 4
F = 16

EPAD = 802816
HALF = 25088
NTAB = 2 * HALF
NPAD = 51200
BHALF = 256
BTAB = 2 * BHALF

_EB = 2048
_NB = 1024


def _sc_gather(table, idx, d, n_items):
  per_w = n_items // (NC * NS)
  nch = per_w // CHUNK
  mesh = plsc.VectorSubcoreMesh(core_axis_name="c", subcore_axis_name="s")

  @functools.partial(
      pl.kernel, mesh=mesh,
      out_type=jax.ShapeDtypeStruct((n_items, d), jnp.float32),
      compiler_params=pltpu.CompilerParams(use_tc_tiling_on_sc=False),
      scratch_types=[
          pltpu.VMEM((CHUNK,), jnp.int32),
          pltpu.VMEM((CHUNK, d), jnp.float32),
          pltpu.SemaphoreType.DMA,
      ],
  )
  def k(table_hbm, idx_hbm, out_hbm, idx_v, rows_v, sem):
    wid = lax.axis_index("s") * NC + lax.axis_index("c")

    def body(j, carry):
      base = wid * per_w + j * CHUNK
      pltpu.sync_copy(idx_hbm.at[pl.ds(base, CHUNK)], idx_v)
      pltpu.async_copy(table_hbm.at[idx_v], rows_v, sem).wait()
      pltpu.sync_copy(rows_v, out_hbm.at[pl.ds(base, CHUNK)])
      return carry

    lax.fori_loop(0, nch, body, 0)

  return k(table, idx)


def _sc_scatter_add(idx2, vals, zeros, d, n_items, half):
  per_s = n_items // NS
  nch = per_s // CHUNK
  rps = half // NS
  mesh = plsc.VectorSubcoreMesh(core_axis_name="c", subcore_axis_name="s")

  @functools.partial(
      pl.kernel, mesh=mesh,
      out_type=jax.ShapeDtypeStruct((2 * half, d), jnp.float32),
      compiler_params=pltpu.CompilerParams(use_tc_tiling_on_sc=False),
      scratch_types=[
          pltpu.VMEM((CHUNK,), jnp.int32),
          pltpu.VMEM((CHUNK, d), jnp.float32),
          pltpu.VMEM_SHARED((half + 16, d), jnp.float32),
      ],
  )
  def k(idx_hbm, vals_hbm, zeros_hbm, out_hbm, idx_v, rows_v, acc_sh):
    cid = lax.axis_index("c")
    sid = lax.axis_index("s")

    @pl.when(sid == 0)
    def _zero():
      pltpu.sync_copy(zeros_hbm, acc_sh)

    plsc.subcore_barrier()

    def body(j, carry):
      base = sid * per_s + j * CHUNK
      pltpu.sync_copy(idx_hbm.at[cid, pl.ds(base, CHUNK)], idx_v)
      pltpu.sync_copy(vals_hbm.at[pl.ds(base, CHUNK)], rows_v)
      pltpu.sync_copy(rows_v, acc_sh.at[idx_v], add=True)
      return carry

    lax.fori_loop(0, nch, body, 0)
    plsc.subcore_barrier()
    pltpu.sync_copy(acc_sh.at[pl.ds(sid * rps, rps)],
                    out_hbm.at[pl.ds(cid * half + sid * rps, rps)])

  return k(idx2, vals, zeros)


def _localize(idx, half, n_items):
  blk = 2048

  def body(i_ref, o0_ref, o1_ref):
    v = i_ref[...]
    in0 = v < half
    o0_ref[...] = jnp.where(in0, v, half)
    o1_ref[...] = jnp.where(in0, half, v - half)

  o0, o1 = pl.pallas_call(
      body,
      grid=(n_items // blk,),
      in_specs=[pl.BlockSpec((blk,), lambda i: (i,))],
      out_specs=[pl.BlockSpec((blk,), lambda i: (i,))] * 2,
      out_shape=[jax.ShapeDtypeStruct((n_items,), jnp.int32)] * 2,
  )(idx)
  return jnp.stack([o0, o1])


def _node_embed(pos_p, num_p, w_species):
  nspec = w_species.shape[0]

  def body(p_ref, n_ref, w_ref, o_ref):
    pos = p_ref[...]
    num = n_ref[...]
    it = lax.broadcasted_iota(jnp.int32, (_NB, nspec), 1)
    oh = (num == it).astype(jnp.float32)
    cn = jnp.dot(oh, w_ref[...], preferred_element_type=jnp.float32)
    o_ref[...] = jnp.concatenate([pos, cn], axis=1)

  return pl.pallas_call(
      body,
      grid=(NTAB // _NB,),
      in_specs=[
          pl.BlockSpec((_NB, 4), lambda i: (i, 0)),
          pl.BlockSpec((_NB, 1), lambda i: (i, 0)),
          pl.BlockSpec((nspec, C), lambda i: (0, 0)),
      ],
      out_specs=pl.BlockSpec((_NB, 8), lambda i: (i, 0)),
      out_shape=jax.ShapeDtypeStruct((NTAB, 8), jnp.float32),
  )(pos_p, num_p, w_species)


def _edge_geom(gs, gd, eo):
  def body(s_ref, d_ref, e_ref, o_ref):
    gs_ = s_ref[...]
    gd_ = d_ref[...]
    eo_ = e_ref[...]
    rij = gd_[:, 0:3] - gs_[:, 0:3] + eo_[:, 0:3]
    d2 = jnp.sum(rij * rij, axis=1, keepdims=True)
    dist = jnp.sqrt(d2 + 1e-12)
    mus = 0.5 + 1.5 * lax.broadcasted_iota(jnp.int32, (1, 4), 1).astype(
        jnp.float32)
    g = jnp.exp(-((dist - mus) ** 2) / 0.25)
    fc = 0.5 * (jnp.cos(jnp.pi * jnp.clip(dist / 5.0, 0.0, 1.0)) + 1.0)
    g = g * fc
    csrc = gs_[:, 4:8]
    ef = csrc[:, :, None] * g[:, None, :]
    o_ref[...] = ef.reshape(_EB, 16)

  return pl.pallas_call(
      body,
      grid=(EPAD // _EB,),
      in_specs=[
          pl.BlockSpec((_EB, 8), lambda i: (i, 0)),
          pl.BlockSpec((_EB, 8), lambda i: (i, 0)),
          pl.BlockSpec((_EB, 4), lambda i: (i, 0)),
      ],
      out_specs=pl.BlockSpec((_EB, 16), lambda i: (i, 0)),
      out_shape=jax.ShapeDtypeStruct((EPAD, 16), jnp.float32),
  )(gs, gd, eo)


def _silu(x):
  return x * (1.0 / (1.0 + jnp.exp(-x)))


def _node_gat_tables(x_in, den, wbd, a_s, a_d, ln_g, ln_b, first):
  def body(x_ref, den_ref, w_ref, as_ref, ad_ref, g_ref, b_ref,
           ts_ref, td_ref):
    if first:
      dens = x_ref[...]
      d3 = dens.reshape(_NB, C, R)
      ps = (d3[:, :, :, None] * d3[:, :, None, :]).reshape(_NB, C, F)
      mu = jnp.mean(ps, axis=2, keepdims=True)
      var = jnp.mean((ps - mu) ** 2, axis=2, keepdims=True)
      psn = (ps - mu) / jnp.sqrt(var + 1e-5)
      psn = psn * g_ref[...][None] + b_ref[...][None]
      x = psn.reshape(_NB, C * F)
    else:
      acc = x_ref[...].reshape(_NB, C, 16)
      den_ = den_ref[...][:, 0:C]
      x = _silu(acc / (den_[:, :, None] + 1e-9)).reshape(_NB, C * 16)
    h = jnp.dot(x, w_ref[...], preferred_element_type=jnp.float32)
    h4 = h.reshape(_NB, C, 16)
    als = jnp.sum(h4 * as_ref[...][None], axis=2)
    ald = jnp.sum(h4 * ad_ref[...][None], axis=2)
    z4 = jnp.zeros((_NB, 4), jnp.float32)
    ts_ref[...] = jnp.concatenate([h, als, z4], axis=1)
    td_ref[...] = jnp.concatenate([ald, z4], axis=1)

  din = 16 if first else 64
  return pl.pallas_call(
      body,
      grid=(NTAB // _NB,),
      in_specs=[
          pl.BlockSpec((_NB, din), lambda i: (i, 0)),
          pl.BlockSpec((_NB, 8), lambda i: (i, 0)),
          pl.BlockSpec((64, 64), lambda i: (0, 0)),
          pl.BlockSpec((C, 16), lambda i: (0, 0)),
          pl.BlockSpec((C, 16), lambda i: (0, 0)),
          pl.BlockSpec((1, F), lambda i: (0, 0)),
          pl.BlockSpec((1, F), lambda i: (0, 0)),
      ],
      out_specs=[
          pl.BlockSpec((_NB, 72), lambda i: (i, 0)),
          pl.BlockSpec((_NB, 8), lambda i: (i, 0)),
      ],
      out_shape=[
          jax.ShapeDtypeStruct((NTAB, 72), jnp.float32),
          jax.ShapeDtypeStruct((NTAB, 8), jnp.float32),
      ],
  )(x_in, den, wbd, a_s, a_d, ln_g, ln_b)


def _edge_att(ts, td):
  def body(s_ref, d_ref, ex_ref, mv_ref):
    ts_ = s_ref[...]
    td_ = d_ref[...]
    e = ts_[:, 64:68] + td_[:, 0:4]
    e = jnp.where(e > 0, e, 0.2 * e)
    ex = jnp.exp(e)
    ex_ref[...] = jnp.concatenate(
        [ex, jnp.zeros((_EB, 4), jnp.float32)], axis=1)
    h4 = ts_[:, 0:64].reshape(_EB, C, 16)
    mv_ref[...] = (ex[:, :, None] * h4).reshape(_EB, 64)

  return pl.pallas_call(
      body,
      grid=(EPAD // _EB,),
      in_specs=[
          pl.BlockSpec((_EB, 72), lambda i: (i, 0)),
          pl.BlockSpec((_EB, 8), lambda i: (i, 0)),
      ],
      out_specs=[
          pl.BlockSpec((_EB, 8), lambda i: (i, 0)),
          pl.BlockSpec((_EB, 64), lambda i: (i, 0)),
      ],
      out_shape=[
          jax.ShapeDtypeStruct((EPAD, 8), jnp.float32),
          jax.ShapeDtypeStruct((EPAD, 64), jnp.float32),
      ],
  )(ts, td)


def _node_head(acc, den, wnbd, wo_flat):
  def body(a_ref, d_ref, w_ref, wo_ref, o_ref):
    acc_ = a_ref[...].reshape(_NB, C, 16)
    den_ = d_ref[...][:, 0:C]
    x = _silu(acc_ / (den_[:, :, None] + 1e-9)).reshape(_NB, 64)
    xh = _silu(jnp.dot(x, w_ref[...], preferred_element_type=jnp.float32))
    y = jnp.sum(xh * wo_ref[...], axis=1, keepdims=True)
    o_ref[...] = jnp.concatenate(
        [y, jnp.zeros((_NB, 7), jnp.float32)], axis=1)

  return pl.pallas_call(
      body,
      grid=(NTAB // _NB,),
      in_specs=[
          pl.BlockSpec((_NB, 64), lambda i: (i, 0)),
          pl.BlockSpec((_NB, 8), lambda i: (i, 0)),
          pl.BlockSpec((64, 64), lambda i: (0, 0)),
          pl.BlockSpec((1, 64), lambda i: (0, 0)),
      ],
      out_specs=pl.BlockSpec((_NB, 8), lambda i: (i, 0)),
      out_shape=jax.ShapeDtypeStruct((NTAB, 8), jnp.float32),
  )(acc, den, wnbd, wo_flat)


def _blockdiag(w):
  c, din, dout = w.shape
  out = jnp.zeros((c * din, c * dout), jnp.float32)
  for i in range(c):
    out = out.at[i * din:(i + 1) * din, i * dout:(i + 1) * dout].set(w[i])
  return out


def kernel(positions, cells, numbers, edge_indices, edge_offsets, batch,
           W_species, ln_gamma, ln_beta, Wc1, a_src1, a_dst1,
           Wc2, a_src2, a_dst2, Wn, Wo):
  f32 = jnp.float32
  i32 = jnp.int32

  src = edge_indices[0].astype(i32)
  dst = edge_indices[1].astype(i32)
  pad_e = EPAD - E
  src_p = jnp.concatenate([src, jnp.full((pad_e,), N, i32)])
  dst_p = jnp.concatenate([dst, jnp.full((pad_e,), N, i32)])
  eo_p = jnp.concatenate(
      [edge_offsets.astype(f32), jnp.zeros((pad_e, 3), f32)], axis=0)
  eo_p = jnp.concatenate([eo_p, jnp.zeros((EPAD, 1), f32)], axis=1)

  pos_p = jnp.concatenate([positions.astype(f32), jnp.zeros((N, 1), f32)],
                          axis=1)
  pos_p = jnp.concatenate([pos_p, jnp.zeros((NTAB - N, 4), f32)], axis=0)
  num_p = jnp.concatenate([numbers.astype(i32), jnp.zeros((NTAB - N,), i32)]
                          )[:, None]

  z8 = jnp.zeros((HALF + 16, 8), f32)
  z16 = jnp.zeros((HALF + 16, 16), f32)
  z64 = jnp.zeros((HALF + 16, 64), f32)
  zb = jnp.zeros((BHALF + 16, 8), f32)

  w1bd = _blockdiag(Wc1.astype(f32))
  w2bd = _blockdiag(Wc2.astype(f32))
  wnbd = _blockdiag(Wn.astype(f32))
  wo_flat = Wo.astype(f32).reshape(1, C * 16)
  ln_g = ln_gamma.astype(f32).reshape(1, F)
  ln_b = ln_beta.astype(f32).reshape(1, F)

  x0 = _node_embed(pos_p, num_p, W_species.astype(f32))
  gs0 = _sc_gather(x0, src_p, 8, EPAD)
  gd0 = _sc_gather(x0, dst_p, 8, EPAD)
  ef = _edge_geom(gs0, gd0, eo_p)
  dix = _localize(dst_p, HALF, EPAD)
  dens = _sc_scatter_add(dix, ef, z16, 16, EPAD, HALF)

  t1s, t1d = _node_gat_tables(dens, dens[:, 0:8], w1bd,
                              a_src1.astype(f32), a_dst1.astype(f32),
                              ln_g, ln_b, True)
  gs1 = _sc_gather(t1s, src_p, 72, EPAD)
  gd1 = _sc_gather(t1d, dst_p, 8, EPAD)
  ex1, mv1 = _edge_att(gs1, gd1)
  den1 = _sc_scatter_add(dix, ex1, z8, 8, EPAD, HALF)
  acc1 = _sc_scatter_add(dix, mv1, z64, 64, EPAD, HALF)

  t2s, t2d = _node_gat_tables(acc1, den1, w2bd,
                              a_src2.astype(f32), a_dst2.astype(f32),
                              ln_g, ln_b, False)
  gs2 = _sc_gather(t2s, src_p, 72, EPAD)
  gd2 = _sc_gather(t2d, dst_p, 8, EPAD)
  ex2, mv2 = _edge_att(gs2, gd2)
  den2 = _sc_scatter_add(dix, ex2, z8, 8, EPAD, HALF)
  acc2 = _sc_scatter_add(dix, mv2, z64, 64, EPAD, HALF)

  feat = _node_head(acc2, den2, wnbd, wo_flat)
  feat_p = jnp.concatenate([feat, jnp.zeros((NPAD - NTAB, 8), f32)], axis=0)
  batch_p = jnp.concatenate(
      [batch.astype(i32), jnp.full((NPAD - N,), BTAB - 1, i32)])
  bix = _localize(batch_p, BHALF, NPAD)
  esum = _sc_scatter_add(bix, feat_p, zb, 8, NPAD, BHALF)
  return esum[:B, 0:1] / jnp.sqrt(float(C))
